# Initial kernel scaffold; baseline (speedup 1.0000x reference)
import jax
import jax.numpy as jnp
from jax.experimental import pallas as pl


def kernel(values, attrib_idx):
    raise NotImplementedError("write your pallas kernel here")


# speedup vs baseline: 3.1971x; 3.1971x over previous
import jax
import jax.numpy as jnp
from jax import lax
from jax.experimental import pallas as pl
from jax.experimental.pallas import tpu as pltpu, tpu_sc as plsc

B, K, A, V = 4096, 26, 100, 128
NC, NS = 2, 16
NW = NC * NS
SPW = B // NW
LANES = 16
VJ = V // LANES


def _body(values_hbm, idx_hbm, out_hbm, idx_v, vals_v, row_v,
          sem_in0, sem_in1, sem_out0, sem_out1):
    c = lax.axis_index("c")
    s = lax.axis_index("s")
    wid = s * NC + c
    base = wid * SPW

    pltpu.sync_copy(idx_hbm.at[pl.ds(base, SPW)], idx_v)

    zero16 = jnp.zeros((LANES,), jnp.float32)
    sem_in = (sem_in0, sem_in1)
    sem_out = (sem_out0, sem_out1)

    def in_cp(d, b):
        return pltpu.make_async_copy(values_hbm.at[b], vals_v.at[d], sem_in[d])

    def out_cp(d, b):
        return pltpu.make_async_copy(row_v.at[d], out_hbm.at[b], sem_out[d])

    in_cp(0, base).start()
    in_cp(1, base + 1).start()

    def step(g, carry):
        for d in range(2):
            si = 2 * g + d
            b = base + si

            @pl.when(si >= 2)
            def _():
                out_cp(d, b - 2).wait()

            def zrow(a, acc):
                for j in range(VJ):
                    row_v[d, a, pl.ds(j * LANES, LANES)] = zero16
                return acc
            lax.fori_loop(0, A, zrow, 0)

            in_cp(d, b).wait()

            iv0 = idx_v[si, pl.ds(0, LANES)]
            iv1 = idx_v[si, pl.ds(K - LANES, LANES)]
            for k in range(K):
                idx = iv0[k] if k < LANES else iv1[k - (K - LANES)]
                for j in range(VJ):
                    row_v[d, idx, pl.ds(j * LANES, LANES)] = (
                        vals_v[d, k, pl.ds(j * LANES, LANES)])

            out_cp(d, b).start()

            @pl.when(si + 2 < SPW)
            def _():
                in_cp(d, b + 2).start()
        return carry

    lax.fori_loop(0, SPW // 2, step, 0)

    out_cp(0, base + SPW - 2).wait()
    out_cp(1, base + SPW - 1).wait()


def kernel(values, attrib_idx):
    idx32 = attrib_idx.astype(jnp.int32)
    mesh = plsc.VectorSubcoreMesh(core_axis_name="c", subcore_axis_name="s")
    run = pl.kernel(
        _body,
        out_type=jax.ShapeDtypeStruct((B, A, V), jnp.float32),
        mesh=mesh,
        scratch_types=[
            pltpu.VMEM((SPW, K), jnp.int32),
            pltpu.VMEM((2, K, V), jnp.float32),
            pltpu.VMEM((2, A, V), jnp.float32),
            pltpu.SemaphoreType.DMA,
            pltpu.SemaphoreType.DMA,
            pltpu.SemaphoreType.DMA,
            pltpu.SemaphoreType.DMA,
        ],
    )
    return run(values, idx32)
